# baseline (speedup 1.0000x reference)
import functools

import jax
import jax.numpy as jnp
from jax.experimental import pallas as pl
from jax.experimental.pallas import tpu as pltpu


def _fused_kernel(x_ref, idxr_ref, wfc_ref, bfc_ref, feat_ref,
                  wms_ref, bms_ref, out_ref,
                  xs_ref, sx_ref, sumsq_ref, cnt_ref, tabs_ref, tabt_ref,
                  *, n_tiles, tile_n):
    i = pl.program_id(0)
    j = pl.program_id(1)
    B = sx_ref.shape[0]
    outC = out_ref.shape[1]

    @pl.when(i == 0)
    def _stats_phase():
        @pl.when(j == 0)
        def _():
            sx_ref[...] = jnp.zeros_like(sx_ref)
            sumsq_ref[...] = jnp.zeros_like(sumsq_ref)
            cnt_ref[...] = jnp.zeros_like(cnt_ref)

        xb = x_ref[...].astype(jnp.bfloat16)
        xs_ref[j] = xb

        y = jnp.dot(xb, wfc_ref[...],
                    preferred_element_type=jnp.float32) + bfc_ref[...]

        gid = jax.lax.broadcasted_iota(jnp.int32, (B, tile_n), 0)
        idxt = idxr_ref[:, pl.ds(j * tile_n, tile_n)]
        ohf = jnp.where(gid == idxt, 1.0, 0.0)
        oh = ohf.astype(jnp.bfloat16)

        sx_ref[...] += jnp.dot(oh, xb, preferred_element_type=jnp.float32)
        sumsq_ref[...] += jnp.dot(oh, (y * y).astype(jnp.bfloat16),
                                  preferred_element_type=jnp.float32)
        cnt_ref[...] += jnp.sum(ohf, axis=1, keepdims=True)

    @pl.when(i == 1)
    def _apply_phase():
        @pl.when(j == 0)
        def _finalize():
            c = cnt_ref[...]
            inv_c = 1.0 / jnp.maximum(c, 1.0)
            sum_y = jnp.dot(sx_ref[...].astype(jnp.bfloat16), wfc_ref[...],
                            preferred_element_type=jnp.float32) + c * bfc_ref[...]
            mean = sum_y * inv_c
            var = jnp.maximum(sumsq_ref[...] * inv_c - mean * mean, 0.0)
            inv_std = jax.lax.rsqrt(var + 1e-14)
            musig = jnp.dot(feat_ref[...], wms_ref[...],
                            preferred_element_type=jnp.float32) + bms_ref[...]
            scale = musig[:, outC:] * inv_std
            shift = musig[:, :outC] - mean * scale
            tabs_ref[...] = scale.astype(jnp.bfloat16)
            tabt_ref[...] = shift.astype(jnp.bfloat16)

        xb = xs_ref[j]
        y = jnp.dot(xb, wfc_ref[...],
                    preferred_element_type=jnp.float32) + bfc_ref[...]

        gid = jax.lax.broadcasted_iota(jnp.int32, (B, tile_n), 0)
        idxt = idxr_ref[:, pl.ds(j * tile_n, tile_n)]
        oh = jnp.where(gid == idxt, 1.0, 0.0).astype(jnp.bfloat16)
        dn = (((0,), (0,)), ((), ()))
        sc = jax.lax.dot_general(oh, tabs_ref[...], dn,
                                 preferred_element_type=jnp.float32)
        sh = jax.lax.dot_general(oh, tabt_ref[...], dn,
                                 preferred_element_type=jnp.float32)
        out_ref[...] = jnp.maximum(y * sc + sh, 0.0)


def kernel(x, origin_feat, idx, wfc, bfc, wmu, bmu, wsig, bsig):
    N, inC = x.shape
    B, featC = origin_feat.shape
    outC = wfc.shape[1]

    tile_n = min(2048, N)
    n_tiles = N // tile_n
    assert N % tile_n == 0

    idx_row = idx.astype(jnp.int32).reshape(1, N)
    wfc_b = wfc.astype(jnp.bfloat16)
    wms = jnp.concatenate([wmu, wsig], axis=1)
    bms = jnp.concatenate([bmu, bsig], axis=1)

    out = pl.pallas_call(
        functools.partial(_fused_kernel, n_tiles=n_tiles, tile_n=tile_n),
        out_shape=jax.ShapeDtypeStruct((N, outC), jnp.float32),
        grid=(2, n_tiles),
        in_specs=[
            pl.BlockSpec((tile_n, inC), lambda i, j: ((1 - i) * j, 0)),
            pl.BlockSpec((1, N), lambda i, j: (0, 0)),
            pl.BlockSpec((inC, outC), lambda i, j: (0, 0)),
            pl.BlockSpec((1, outC), lambda i, j: (0, 0)),
            pl.BlockSpec((B, featC), lambda i, j: (0, 0)),
            pl.BlockSpec((featC, 2 * outC), lambda i, j: (0, 0)),
            pl.BlockSpec((1, 2 * outC), lambda i, j: (0, 0)),
        ],
        out_specs=pl.BlockSpec((tile_n, outC), lambda i, j: (i * j, 0)),
        scratch_shapes=[
            pltpu.VMEM((n_tiles, tile_n, inC), jnp.bfloat16),
            pltpu.VMEM((B, inC), jnp.float32),
            pltpu.VMEM((B, outC), jnp.float32),
            pltpu.VMEM((B, 1), jnp.float32),
            pltpu.VMEM((B, outC), jnp.bfloat16),
            pltpu.VMEM((B, outC), jnp.bfloat16),
        ],
        compiler_params=pltpu.CompilerParams(
            dimension_semantics=("arbitrary", "arbitrary"),
            vmem_limit_bytes=60 * 1024 * 1024),
    )(x, idx_row, wfc_b, bfc, origin_feat, wms, bms)

    return out


# speedup vs baseline: 1.2690x; 1.2690x over previous
import functools

import jax
import jax.numpy as jnp
from jax.experimental import pallas as pl
from jax.experimental.pallas import tpu as pltpu


def _fused_kernel(x_ref, idxr_ref, wfc_ref, bfc_ref, feat_ref,
                  wms_ref, bms_ref, out_ref,
                  xs_ref, sx_ref, sumsq_ref, cnt_ref, tabs_ref, tabt_ref,
                  *, n_tiles, tile_n):
    i = pl.program_id(0)
    j = pl.program_id(1)
    B = sx_ref.shape[0]
    outC = out_ref.shape[1]

    @pl.when(i == 0)
    def _stats_phase():
        @pl.when(j == 0)
        def _():
            sx_ref[...] = jnp.zeros_like(sx_ref)
            sumsq_ref[...] = jnp.zeros_like(sumsq_ref)
            cnt_ref[...] = jnp.zeros_like(cnt_ref)

        xb = x_ref[...].astype(jnp.bfloat16)
        xs_ref[j] = xb

        y = jnp.dot(xb, wfc_ref[...],
                    preferred_element_type=jnp.float32) + bfc_ref[...]

        gid = jax.lax.broadcasted_iota(jnp.int32, (B, tile_n), 0)
        idxt = idxr_ref[:, pl.ds(j * tile_n, tile_n)]
        ohf = jnp.where(gid == idxt, 1.0, 0.0)
        oh = ohf.astype(jnp.bfloat16)

        sx_ref[...] += jnp.dot(oh, xb, preferred_element_type=jnp.float32)
        sumsq_ref[...] += jnp.dot(oh, (y * y).astype(jnp.bfloat16),
                                  preferred_element_type=jnp.float32)
        cnt_ref[...] += jnp.sum(ohf, axis=1, keepdims=True)

        @pl.when(j == n_tiles - 1)
        def _finalize():
            c = cnt_ref[...]
            inv_c = 1.0 / jnp.maximum(c, 1.0)
            sum_y = jnp.dot(sx_ref[...].astype(jnp.bfloat16), wfc_ref[...],
                            preferred_element_type=jnp.float32) + c * bfc_ref[...]
            mean = sum_y * inv_c
            var = jnp.maximum(sumsq_ref[...] * inv_c - mean * mean, 0.0)
            inv_std = jax.lax.rsqrt(var + 1e-14)
            musig = jnp.dot(feat_ref[...], wms_ref[...],
                            preferred_element_type=jnp.float32) + bms_ref[...]
            scale = musig[:, outC:] * inv_std
            shift = musig[:, :outC] - mean * scale
            tabs_ref[...] = scale.astype(jnp.bfloat16)
            tabt_ref[...] = shift.astype(jnp.bfloat16)

    @pl.when(i == 1)
    def _apply_phase():
        xb = xs_ref[j]
        y = jnp.dot(xb, wfc_ref[...],
                    preferred_element_type=jnp.float32) + bfc_ref[...]

        gid = jax.lax.broadcasted_iota(jnp.int32, (B, tile_n), 0)
        idxt = idxr_ref[:, pl.ds(j * tile_n, tile_n)]
        oh = jnp.where(gid == idxt, 1.0, 0.0).astype(jnp.bfloat16)
        dn = (((0,), (0,)), ((), ()))
        sc = jax.lax.dot_general(oh, tabs_ref[...], dn,
                                 preferred_element_type=jnp.float32)
        sh = jax.lax.dot_general(oh, tabt_ref[...], dn,
                                 preferred_element_type=jnp.float32)
        out_ref[...] = jnp.maximum(y * sc + sh, 0.0)


def kernel(x, origin_feat, idx, wfc, bfc, wmu, bmu, wsig, bsig):
    N, inC = x.shape
    B, featC = origin_feat.shape
    outC = wfc.shape[1]

    tile_n = min(4096, N)
    n_tiles = N // tile_n
    assert N % tile_n == 0

    idx_row = idx.astype(jnp.int32).reshape(1, N)
    wfc_b = wfc.astype(jnp.bfloat16)
    wms = jnp.concatenate([wmu, wsig], axis=1)
    bms = jnp.concatenate([bmu, bsig], axis=1)

    out = pl.pallas_call(
        functools.partial(_fused_kernel, n_tiles=n_tiles, tile_n=tile_n),
        out_shape=jax.ShapeDtypeStruct((N, outC), jnp.float32),
        grid=(2, n_tiles),
        in_specs=[
            pl.BlockSpec((tile_n, inC), lambda i, j: ((1 - i) * j, 0)),
            pl.BlockSpec((1, N), lambda i, j: (0, 0)),
            pl.BlockSpec((inC, outC), lambda i, j: (0, 0)),
            pl.BlockSpec((1, outC), lambda i, j: (0, 0)),
            pl.BlockSpec((B, featC), lambda i, j: (0, 0)),
            pl.BlockSpec((featC, 2 * outC), lambda i, j: (0, 0)),
            pl.BlockSpec((1, 2 * outC), lambda i, j: (0, 0)),
        ],
        out_specs=pl.BlockSpec((tile_n, outC), lambda i, j: (i * j, 0)),
        scratch_shapes=[
            pltpu.VMEM((n_tiles, tile_n, inC), jnp.bfloat16),
            pltpu.VMEM((B, inC), jnp.float32),
            pltpu.VMEM((B, outC), jnp.float32),
            pltpu.VMEM((B, 1), jnp.float32),
            pltpu.VMEM((B, outC), jnp.bfloat16),
            pltpu.VMEM((B, outC), jnp.bfloat16),
        ],
        compiler_params=pltpu.CompilerParams(
            dimension_semantics=("arbitrary", "arbitrary"),
            vmem_limit_bytes=63 * 1024 * 1024),
    )(x, idx_row, wfc_b, bfc, origin_feat, wms, bms)

    return out


# speedup vs baseline: 1.3192x; 1.0395x over previous
import functools

import jax
import jax.numpy as jnp
from jax.experimental import pallas as pl
from jax.experimental.pallas import tpu as pltpu


def _fused_kernel(x_ref, idxr_ref, wfc_ref, bfc_ref, feat_ref,
                  wms_ref, bms_ref, out_ref,
                  xs_ref, sx_ref, sumsq_ref, cnt_ref, tabs_ref, tabt_ref,
                  *, n_tiles, tile_n):
    s = pl.program_id(0)
    j = s
    k = s - n_tiles
    B = sx_ref.shape[0]
    outC = out_ref.shape[1]

    @pl.when(s < n_tiles)
    def _stats_phase():
        @pl.when(j == 0)
        def _():
            sx_ref[...] = jnp.zeros_like(sx_ref)
            sumsq_ref[...] = jnp.zeros_like(sumsq_ref)
            cnt_ref[...] = jnp.zeros_like(cnt_ref)

        xb = x_ref[...].astype(jnp.bfloat16)
        xs_ref[j] = xb

        y = jnp.dot(xb, wfc_ref[...],
                    preferred_element_type=jnp.float32) + bfc_ref[...]

        gid = jax.lax.broadcasted_iota(jnp.int32, (B, tile_n), 0)
        idxt = idxr_ref[:, pl.ds(j * tile_n, tile_n)]
        ohf = jnp.where(gid == idxt, 1.0, 0.0)
        oh = ohf.astype(jnp.bfloat16)

        sx_ref[...] += jnp.dot(oh, xb, preferred_element_type=jnp.float32)
        sumsq_ref[...] += jnp.dot(oh, (y * y).astype(jnp.bfloat16),
                                  preferred_element_type=jnp.float32)
        cnt_ref[...] += jnp.sum(ohf, axis=1, keepdims=True)

        @pl.when(j == n_tiles - 1)
        def _finalize():
            c = cnt_ref[...]
            inv_c = 1.0 / jnp.maximum(c, 1.0)
            sum_y = jnp.dot(sx_ref[...].astype(jnp.bfloat16), wfc_ref[...],
                            preferred_element_type=jnp.float32) + c * bfc_ref[...]
            mean = sum_y * inv_c
            var = jnp.maximum(sumsq_ref[...] * inv_c - mean * mean, 0.0)
            inv_std = jax.lax.rsqrt(var + 1e-14)
            musig = jnp.dot(feat_ref[...], wms_ref[...],
                            preferred_element_type=jnp.float32) + bms_ref[...]
            scale = musig[:, outC:] * inv_std
            shift = musig[:, :outC] - mean * scale
            tabs_ref[...] = scale.astype(jnp.bfloat16)
            tabt_ref[...] = shift.astype(jnp.bfloat16)

    @pl.when(s >= n_tiles)
    def _apply_phase():
        dn = (((0,), (0,)), ((), ()))
        for h in range(2):
            xb = xs_ref[2 * k + h]
            y = jnp.dot(xb, wfc_ref[...],
                        preferred_element_type=jnp.float32) + bfc_ref[...]
            gid = jax.lax.broadcasted_iota(jnp.int32, (B, tile_n), 0)
            idxt = idxr_ref[:, pl.ds((2 * k + h) * tile_n, tile_n)]
            oh = jnp.where(gid == idxt, 1.0, 0.0).astype(jnp.bfloat16)
            sc = jax.lax.dot_general(oh, tabs_ref[...], dn,
                                     preferred_element_type=jnp.float32)
            sh = jax.lax.dot_general(oh, tabt_ref[...], dn,
                                     preferred_element_type=jnp.float32)
            out_ref[pl.ds(h * tile_n, tile_n), :] = jnp.maximum(y * sc + sh, 0.0)


def kernel(x, origin_feat, idx, wfc, bfc, wmu, bmu, wsig, bsig):
    N, inC = x.shape
    B, featC = origin_feat.shape
    outC = wfc.shape[1]

    tile_n = min(4096, N)
    n_tiles = N // tile_n
    assert N % tile_n == 0

    idx_row = idx.astype(jnp.int32).reshape(1, N)
    wfc_b = wfc.astype(jnp.bfloat16)
    wms = jnp.concatenate([wmu, wsig], axis=1)
    bms = jnp.concatenate([bmu, bsig], axis=1)

    out = pl.pallas_call(
        functools.partial(_fused_kernel, n_tiles=n_tiles, tile_n=tile_n),
        out_shape=jax.ShapeDtypeStruct((N, outC), jnp.float32),
        grid=(n_tiles + n_tiles // 2,),
        in_specs=[
            pl.BlockSpec((tile_n, inC),
                         lambda s: (jnp.where(s < n_tiles, s, 0), 0)),
            pl.BlockSpec((1, N), lambda s: (0, 0)),
            pl.BlockSpec((inC, outC), lambda s: (0, 0)),
            pl.BlockSpec((1, outC), lambda s: (0, 0)),
            pl.BlockSpec((B, featC), lambda s: (0, 0)),
            pl.BlockSpec((featC, 2 * outC), lambda s: (0, 0)),
            pl.BlockSpec((1, 2 * outC), lambda s: (0, 0)),
        ],
        out_specs=pl.BlockSpec((2 * tile_n, outC),
                               lambda s: (jnp.where(s < n_tiles, 0, s - n_tiles), 0)),
        scratch_shapes=[
            pltpu.VMEM((n_tiles, tile_n, inC), jnp.bfloat16),
            pltpu.VMEM((B, inC), jnp.float32),
            pltpu.VMEM((B, outC), jnp.float32),
            pltpu.VMEM((B, 1), jnp.float32),
            pltpu.VMEM((B, outC), jnp.bfloat16),
            pltpu.VMEM((B, outC), jnp.bfloat16),
        ],
        compiler_params=pltpu.CompilerParams(
            dimension_semantics=("arbitrary",),
            vmem_limit_bytes=63 * 1024 * 1024),
    )(x, idx_row, wfc_b, bfc, origin_feat, wms, bms)

    return out


# speedup vs baseline: 1.3931x; 1.0561x over previous
import functools

import jax
import jax.numpy as jnp
from jax.experimental import pallas as pl
from jax.experimental.pallas import tpu as pltpu


def _fused_kernel(x_ref, idxr_ref, wfc_ref, bfc_ref, feat_ref,
                  wms_ref, bms_ref, out_ref,
                  xs_ref, sx_ref, sumsq_ref, cnt_ref, tabs_ref, tabt_ref,
                  *, n_tiles, tile_n):
    n_steps0 = n_tiles // 2
    s = pl.program_id(0)
    j = s
    k = s - n_steps0
    B = sx_ref.shape[0]
    outC = out_ref.shape[1]

    @pl.when(s < n_steps0)
    def _stats_phase():
        @pl.when(j == 0)
        def _():
            sx_ref[...] = jnp.zeros_like(sx_ref)
            sumsq_ref[...] = jnp.zeros_like(sumsq_ref)
            cnt_ref[...] = jnp.zeros_like(cnt_ref)

        for h in range(2):
            xb = x_ref[pl.ds(h * tile_n, tile_n), :].astype(jnp.bfloat16)
            xs_ref[2 * j + h] = xb

            y = jnp.dot(xb, wfc_ref[...],
                        preferred_element_type=jnp.float32) + bfc_ref[...]

            gid = jax.lax.broadcasted_iota(jnp.int32, (B, tile_n), 0)
            idxt = idxr_ref[:, pl.ds((2 * j + h) * tile_n, tile_n)]
            ohf = jnp.where(gid == idxt, 1.0, 0.0)
            oh = ohf.astype(jnp.bfloat16)

            sx_ref[...] += jnp.dot(oh, xb, preferred_element_type=jnp.float32)
            sumsq_ref[...] += jnp.dot(oh, (y * y).astype(jnp.bfloat16),
                                      preferred_element_type=jnp.float32)
            cnt_ref[...] += jnp.sum(ohf, axis=1, keepdims=True)

        @pl.when(j == n_steps0 - 1)
        def _finalize():
            c = cnt_ref[...]
            inv_c = 1.0 / jnp.maximum(c, 1.0)
            sum_y = jnp.dot(sx_ref[...].astype(jnp.bfloat16), wfc_ref[...],
                            preferred_element_type=jnp.float32) + c * bfc_ref[...]
            mean = sum_y * inv_c
            var = jnp.maximum(sumsq_ref[...] * inv_c - mean * mean, 0.0)
            inv_std = jax.lax.rsqrt(var + 1e-14)
            musig = jnp.dot(feat_ref[...], wms_ref[...],
                            preferred_element_type=jnp.float32) + bms_ref[...]
            scale = musig[:, outC:] * inv_std
            shift = musig[:, :outC] - mean * scale
            tabs_ref[...] = scale.astype(jnp.bfloat16)
            tabt_ref[...] = shift.astype(jnp.bfloat16)

    @pl.when(s >= n_steps0)
    def _apply_phase():
        dn = (((0,), (0,)), ((), ()))
        for h in range(2):
            xb = xs_ref[2 * k + h]
            y = jnp.dot(xb, wfc_ref[...],
                        preferred_element_type=jnp.float32) + bfc_ref[...]
            gid = jax.lax.broadcasted_iota(jnp.int32, (B, tile_n), 0)
            idxt = idxr_ref[:, pl.ds((2 * k + h) * tile_n, tile_n)]
            oh = jnp.where(gid == idxt, 1.0, 0.0).astype(jnp.bfloat16)
            sc = jax.lax.dot_general(oh, tabs_ref[...], dn,
                                     preferred_element_type=jnp.float32)
            sh = jax.lax.dot_general(oh, tabt_ref[...], dn,
                                     preferred_element_type=jnp.float32)
            out_ref[pl.ds(h * tile_n, tile_n), :] = jnp.maximum(y * sc + sh, 0.0)


def kernel(x, origin_feat, idx, wfc, bfc, wmu, bmu, wsig, bsig):
    N, inC = x.shape
    B, featC = origin_feat.shape
    outC = wfc.shape[1]

    tile_n = min(4096, N)
    n_tiles = N // tile_n
    assert N % tile_n == 0

    idx_row = idx.astype(jnp.int32).reshape(1, N)
    wfc_b = wfc.astype(jnp.bfloat16)
    wms = jnp.concatenate([wmu, wsig], axis=1)
    bms = jnp.concatenate([bmu, bsig], axis=1)

    out = pl.pallas_call(
        functools.partial(_fused_kernel, n_tiles=n_tiles, tile_n=tile_n),
        out_shape=jax.ShapeDtypeStruct((N, outC), jnp.float32),
        grid=(n_tiles,),
        in_specs=[
            pl.BlockSpec((2 * tile_n, inC),
                         lambda s: (jnp.where(s < n_tiles // 2, s, 0), 0)),
            pl.BlockSpec((1, N), lambda s: (0, 0)),
            pl.BlockSpec((inC, outC), lambda s: (0, 0)),
            pl.BlockSpec((1, outC), lambda s: (0, 0)),
            pl.BlockSpec((B, featC), lambda s: (0, 0)),
            pl.BlockSpec((featC, 2 * outC), lambda s: (0, 0)),
            pl.BlockSpec((1, 2 * outC), lambda s: (0, 0)),
        ],
        out_specs=pl.BlockSpec((2 * tile_n, outC),
                               lambda s: (jnp.where(s < n_tiles // 2, 0,
                                                    s - n_tiles // 2), 0)),
        scratch_shapes=[
            pltpu.VMEM((n_tiles, tile_n, inC), jnp.bfloat16),
            pltpu.VMEM((B, inC), jnp.float32),
            pltpu.VMEM((B, outC), jnp.float32),
            pltpu.VMEM((B, 1), jnp.float32),
            pltpu.VMEM((B, outC), jnp.bfloat16),
            pltpu.VMEM((B, outC), jnp.bfloat16),
        ],
        compiler_params=pltpu.CompilerParams(
            dimension_semantics=("arbitrary",),
            vmem_limit_bytes=63 * 1024 * 1024),
    )(x, idx_row, wfc_b, bfc, origin_feat, wms, bms)

    return out
